# baseline (speedup 1.0000x reference)
import jax
import jax.numpy as jnp
from jax.experimental import pallas as pl
from jax.experimental.pallas import tpu as pltpu


def _ffn_body(x_ref, af_ref, w1_ref, b1_ref, w2_ref, b2_ref,
              wr1x_ref, wr1a_ref, br1_ref, wr2_ref, br2_ref, out_ref,
              h_ref):
    k = pl.program_id(1)
    nk1 = pl.num_programs(1)
    nk = nk1 - 1
    c0 = jnp.float32(0.7978845608028654)
    c1 = jnp.float32(0.044715)

    def _produce():
        h = jnp.dot(x_ref[...], w1_ref[...],
                    preferred_element_type=jnp.float32)
        h = h + b1_ref[...]
        inner = c0 * (h + c1 * h * h * h)
        h = jnp.float32(0.5) * h * (jnp.float32(1.0) + jnp.tanh(inner))
        return h.astype(h_ref.dtype)

    def _consume():
        return jnp.dot(h_ref[(k - 1) % 2], w2_ref[...],
                       preferred_element_type=jnp.float32)

    @pl.when(k == 0)
    def _prologue():
        h_ref[0] = _produce()

    @pl.when(jnp.logical_and(k > 0, k < nk))
    def _steady():
        h = _produce()
        part = _consume()
        h_ref[k % 2] = h

        @pl.when(k == 1)
        def _init():
            out_ref[...] = part

        @pl.when(k > 1)
        def _accum():
            out_ref[...] += part

    @pl.when(k == nk)
    def _finalize():
        part = _consume()

        @pl.when(nk == 1)
        def _init():
            out_ref[...] = part

        @pl.when(nk > 1)
        def _accum():
            out_ref[...] += part
        g = jnp.dot(x_ref[...], wr1x_ref[...],
                    preferred_element_type=jnp.float32)
        g = g + af_ref[...] * wr1a_ref[...]
        g = g + br1_ref[...]
        g = jnp.float32(0.5) * g * (
            jnp.float32(1.0) + jax.lax.erf(g * jnp.float32(0.7071067811865476)))
        logits = jnp.dot(g, wr2_ref[...],
                         preferred_element_type=jnp.float32) + br2_ref[...]
        probs = jax.nn.sigmoid(logits)
        out_ref[...] = (out_ref[...] + b2_ref[...]) * probs


def kernel(x, attn_feat, w1, b1, w2, b2, wr1, br1, wr2, br2):
    B, S, H = x.shape
    N = B * S
    DFF = w1.shape[1]
    GH = wr1.shape[1]
    cdtype = jnp.bfloat16

    xr = x.reshape(N, H).astype(cdtype)
    af = attn_feat.reshape(N, 1)
    wr1x = wr1[:H].astype(cdtype)
    wr1a = wr1[H:H + 1]
    b1r = b1.reshape(1, DFF)
    b2r = b2.reshape(1, H)
    br1r = br1.reshape(1, GH)
    br2r = br2.reshape(1, 1)

    bm = min(512, N)
    bk = min(2048, DFF)
    nk = DFF // bk
    grid = (N // bm, nk + 1)

    def w1_idx(i, k):
        return (0, jnp.minimum(k, nk - 1))

    def b1_idx(i, k):
        return (0, jnp.minimum(k, nk - 1))

    def w2_idx(i, k):
        return (jnp.maximum(k - 1, 0), 0)

    out = pl.pallas_call(
        _ffn_body,
        grid=grid,
        in_specs=[
            pl.BlockSpec((bm, H), lambda i, k: (i, 0)),
            pl.BlockSpec((bm, 1), lambda i, k: (i, 0)),
            pl.BlockSpec((H, bk), w1_idx),
            pl.BlockSpec((1, bk), b1_idx),
            pl.BlockSpec((bk, H), w2_idx),
            pl.BlockSpec((1, H), lambda i, k: (0, 0)),
            pl.BlockSpec((H, GH), lambda i, k: (0, 0)),
            pl.BlockSpec((1, GH), lambda i, k: (0, 0)),
            pl.BlockSpec((1, GH), lambda i, k: (0, 0)),
            pl.BlockSpec((GH, 1), lambda i, k: (0, 0)),
            pl.BlockSpec((1, 1), lambda i, k: (0, 0)),
        ],
        out_specs=pl.BlockSpec((bm, H), lambda i, k: (i, 0)),
        out_shape=jax.ShapeDtypeStruct((N, H), jnp.float32),
        scratch_shapes=[pltpu.VMEM((2, bm, bk), cdtype)],
        compiler_params=pltpu.CompilerParams(
            dimension_semantics=("parallel", "arbitrary")),
    )(xr, af, w1.astype(cdtype), b1r, w2.astype(cdtype), b2r,
      wr1x, wr1a, br1r, wr2, br2r)

    return out.reshape(B, S, H)


# speedup vs baseline: 1.1036x; 1.1036x over previous
import jax
import jax.numpy as jnp
from jax.experimental import pallas as pl
from jax.experimental.pallas import tpu as pltpu


def _ffn_body(x_ref, af_ref, w1_ref, b1_ref, w2_ref, b2_ref,
              wr1x_ref, wr1a_ref, br1_ref, wr2_ref, br2_ref, out_ref):
    k = pl.program_id(1)
    nk = pl.num_programs(1)

    x = x_ref[...].astype(jnp.bfloat16)
    bk = w1_ref.shape[1]
    nsub = 2
    sk = bk // nsub
    c0 = jnp.float32(0.7978845608028654)
    c0c1 = jnp.float32(0.7978845608028654 * 0.044715)

    parts = []
    for s in range(nsub):
        sl = pl.ds(s * sk, sk)
        h = jnp.dot(x, w1_ref[:, sl], preferred_element_type=jnp.float32)
        h = h + b1_ref[:, sl]
        u = h * (c0 + c0c1 * (h * h))
        ph = jnp.float32(0.5) * h
        g = ph + ph * jnp.tanh(u)
        parts.append(jnp.dot(g.astype(jnp.bfloat16), w2_ref[sl, :],
                             preferred_element_type=jnp.float32))

    part = sum(parts[1:], parts[0])

    @pl.when(k == 0)
    def _init():
        out_ref[...] = part

    @pl.when(k != 0)
    def _accum():
        out_ref[...] += part

    @pl.when(k == nk - 1)
    def _finalize():
        g = jnp.dot(x, wr1x_ref[...], preferred_element_type=jnp.float32)
        g = g + af_ref[...] * wr1a_ref[...]
        g = g + br1_ref[...]
        g = jnp.float32(0.5) * g * (
            jnp.float32(1.0) + jax.lax.erf(g * jnp.float32(0.7071067811865476)))
        logits = jnp.dot(g, wr2_ref[...],
                         preferred_element_type=jnp.float32) + br2_ref[...]
        probs = jax.nn.sigmoid(logits)
        out_ref[...] = (out_ref[...] + b2_ref[...]) * probs


def kernel(x, attn_feat, w1, b1, w2, b2, wr1, br1, wr2, br2):
    B, S, H = x.shape
    N = B * S
    DFF = w1.shape[1]
    GH = wr1.shape[1]
    cdtype = jnp.bfloat16

    xr = x.reshape(N, H)
    af = attn_feat.reshape(N, 1)
    wr1x = wr1[:H].astype(cdtype)
    wr1a = wr1[H:H + 1]
    b1r = b1.reshape(1, DFF)
    b2r = b2.reshape(1, H)
    br1r = br1.reshape(1, GH)
    br2r = br2.reshape(1, 1)

    bm = min(512, N)
    bk = min(2048, DFF)
    grid = (N // bm, DFF // bk)

    out = pl.pallas_call(
        _ffn_body,
        grid=grid,
        in_specs=[
            pl.BlockSpec((bm, H), lambda i, k: (i, 0)),
            pl.BlockSpec((bm, 1), lambda i, k: (i, 0)),
            pl.BlockSpec((H, bk), lambda i, k: (0, k)),
            pl.BlockSpec((1, bk), lambda i, k: (0, k)),
            pl.BlockSpec((bk, H), lambda i, k: (k, 0)),
            pl.BlockSpec((1, H), lambda i, k: (0, 0)),
            pl.BlockSpec((H, GH), lambda i, k: (0, 0)),
            pl.BlockSpec((1, GH), lambda i, k: (0, 0)),
            pl.BlockSpec((1, GH), lambda i, k: (0, 0)),
            pl.BlockSpec((GH, 1), lambda i, k: (0, 0)),
            pl.BlockSpec((1, 1), lambda i, k: (0, 0)),
        ],
        out_specs=pl.BlockSpec((bm, H), lambda i, k: (i, 0)),
        out_shape=jax.ShapeDtypeStruct((N, H), jnp.float32),
        compiler_params=pltpu.CompilerParams(
            dimension_semantics=("parallel", "arbitrary")),
    )(xr, af, w1.astype(cdtype), b1r, w2.astype(cdtype), b2r,
      wr1x, wr1a, br1r, wr2, br2r)

    return out.reshape(B, S, H)


# speedup vs baseline: 1.1048x; 1.0011x over previous
import jax
import jax.numpy as jnp
from jax.experimental import pallas as pl
from jax.experimental.pallas import tpu as pltpu


def _ffn_body(x_ref, af_ref, w1_ref, b1_ref, w2_ref, b2_ref,
              wr1x_ref, wr1a_ref, br1_ref, wr2_ref, br2_ref, out_ref):
    k = pl.program_id(1)
    nk = pl.num_programs(1)

    x = x_ref[...].astype(jnp.bfloat16)
    bk = w1_ref.shape[1]
    nsub = 1
    sk = bk // nsub
    c0 = jnp.float32(0.7978845608028654)
    c0c1 = jnp.float32(0.7978845608028654 * 0.044715)

    parts = []
    for s in range(nsub):
        sl = pl.ds(s * sk, sk)
        h = jnp.dot(x, w1_ref[:, sl], preferred_element_type=jnp.float32)
        h = h + b1_ref[:, sl]
        u = h * (c0 + c0c1 * (h * h))
        ph = jnp.float32(0.5) * h
        g = ph + ph * jnp.tanh(u)
        parts.append(jnp.dot(g.astype(jnp.bfloat16), w2_ref[sl, :],
                             preferred_element_type=jnp.float32))

    part = sum(parts[1:], parts[0])

    @pl.when(k == 0)
    def _init():
        out_ref[...] = part

    @pl.when(k != 0)
    def _accum():
        out_ref[...] += part

    @pl.when(k == nk - 1)
    def _finalize():
        g = jnp.dot(x, wr1x_ref[...], preferred_element_type=jnp.float32)
        g = g + af_ref[...] * wr1a_ref[...]
        g = g + br1_ref[...]
        g = jnp.float32(0.5) * g * (
            jnp.float32(1.0) + jax.lax.erf(g * jnp.float32(0.7071067811865476)))
        logits = jnp.dot(g, wr2_ref[...],
                         preferred_element_type=jnp.float32) + br2_ref[...]
        probs = jax.nn.sigmoid(logits)
        out_ref[...] = (out_ref[...] + b2_ref[...]) * probs


def kernel(x, attn_feat, w1, b1, w2, b2, wr1, br1, wr2, br2):
    B, S, H = x.shape
    N = B * S
    DFF = w1.shape[1]
    GH = wr1.shape[1]
    cdtype = jnp.bfloat16

    xr = x.reshape(N, H)
    af = attn_feat.reshape(N, 1)
    wr1x = wr1[:H].astype(cdtype)
    wr1a = wr1[H:H + 1]
    b1r = b1.reshape(1, DFF)
    b2r = b2.reshape(1, H)
    br1r = br1.reshape(1, GH)
    br2r = br2.reshape(1, 1)

    bm = min(512, N)
    bk = min(2048, DFF)
    grid = (N // bm, DFF // bk)

    out = pl.pallas_call(
        _ffn_body,
        grid=grid,
        in_specs=[
            pl.BlockSpec((bm, H), lambda i, k: (i, 0)),
            pl.BlockSpec((bm, 1), lambda i, k: (i, 0)),
            pl.BlockSpec((H, bk), lambda i, k: (0, k)),
            pl.BlockSpec((1, bk), lambda i, k: (0, k)),
            pl.BlockSpec((bk, H), lambda i, k: (k, 0)),
            pl.BlockSpec((1, H), lambda i, k: (0, 0)),
            pl.BlockSpec((H, GH), lambda i, k: (0, 0)),
            pl.BlockSpec((1, GH), lambda i, k: (0, 0)),
            pl.BlockSpec((1, GH), lambda i, k: (0, 0)),
            pl.BlockSpec((GH, 1), lambda i, k: (0, 0)),
            pl.BlockSpec((1, 1), lambda i, k: (0, 0)),
        ],
        out_specs=pl.BlockSpec((bm, H), lambda i, k: (i, 0)),
        out_shape=jax.ShapeDtypeStruct((N, H), jnp.float32),
        compiler_params=pltpu.CompilerParams(
            dimension_semantics=("parallel", "arbitrary")),
    )(xr, af, w1.astype(cdtype), b1r, w2.astype(cdtype), b2r,
      wr1x, wr1a, br1r, wr2, br2r)

    return out.reshape(B, S, H)


# speedup vs baseline: 1.1123x; 1.0069x over previous
import jax
import jax.numpy as jnp
from jax.experimental import pallas as pl
from jax.experimental.pallas import tpu as pltpu


def kernel(x, attn_feat, w1, b1, w2, b2, wr1, br1, wr2, br2):
    B, S, H = x.shape
    N = B * S
    DFF = w1.shape[1]
    GH = wr1.shape[1]
    cdtype = jnp.bfloat16

    xr = x.reshape(N, H)
    af = attn_feat.reshape(N, 1)
    wr1x = wr1[:H].astype(cdtype)
    wr1a = wr1[H:H + 1]
    b1r = b1.reshape(1, DFF)
    b2r = b2.reshape(1, H)
    br1r = br1.reshape(1, GH)
    br2r = br2.reshape(1, 1)

    bm = min(512, N)
    bk = min(2048, DFF)
    nk = DFF // bk
    grid = (N // bm, nk)

    def body(x_ref, af_ref, w1_ref, b1_ref, w2_ref, b2_ref,
             wr1x_ref, wr1a_ref, br1_ref, wr2_ref, br2_ref, out_ref, p_ref):
        k = pl.program_id(1)
        x_bf = x_ref[...].astype(jnp.bfloat16)
        c0 = jnp.float32(0.7978845608028654)
        c0c1 = jnp.float32(0.7978845608028654 * 0.044715)

        h = jnp.dot(x_bf, w1_ref[...], preferred_element_type=jnp.float32)
        h = h + b1_ref[...]
        u = h * (c0 + c0c1 * (h * h))
        ph = jnp.float32(0.5) * h
        g = ph + ph * jnp.tanh(u)
        part = jnp.dot(g.astype(jnp.bfloat16), w2_ref[...],
                       preferred_element_type=jnp.float32)

        @pl.when(k == 0)
        def _init():
            r = jnp.dot(x_bf, wr1x_ref[...],
                        preferred_element_type=jnp.float32)
            r = r + af_ref[...] * wr1a_ref[...]
            r = r + br1_ref[...]
            r = jnp.float32(0.5) * r * (
                jnp.float32(1.0)
                + jax.lax.erf(r * jnp.float32(0.7071067811865476)))
            logits = jnp.dot(r, wr2_ref[...],
                             preferred_element_type=jnp.float32) + br2_ref[...]
            p_ref[...] = jax.nn.sigmoid(logits)
            if nk == 1:
                out_ref[...] = (part + b2_ref[...]) * p_ref[...]
            else:
                out_ref[...] = part

        if nk > 1:
            @pl.when(jnp.logical_and(k > 0, k < nk - 1))
            def _accum():
                out_ref[...] += part

            @pl.when(k == nk - 1)
            def _finalize():
                out_ref[...] = (out_ref[...] + part + b2_ref[...]) * p_ref[...]

    out = pl.pallas_call(
        body,
        grid=grid,
        in_specs=[
            pl.BlockSpec((bm, H), lambda i, k: (i, 0)),
            pl.BlockSpec((bm, 1), lambda i, k: (i, 0)),
            pl.BlockSpec((H, bk), lambda i, k: (0, k)),
            pl.BlockSpec((1, bk), lambda i, k: (0, k)),
            pl.BlockSpec((bk, H), lambda i, k: (k, 0)),
            pl.BlockSpec((1, H), lambda i, k: (0, 0)),
            pl.BlockSpec((H, GH), lambda i, k: (0, 0)),
            pl.BlockSpec((1, GH), lambda i, k: (0, 0)),
            pl.BlockSpec((1, GH), lambda i, k: (0, 0)),
            pl.BlockSpec((GH, 1), lambda i, k: (0, 0)),
            pl.BlockSpec((1, 1), lambda i, k: (0, 0)),
        ],
        out_specs=pl.BlockSpec((bm, H), lambda i, k: (i, 0)),
        out_shape=jax.ShapeDtypeStruct((N, H), jnp.float32),
        scratch_shapes=[pltpu.VMEM((bm, 1), jnp.float32)],
        compiler_params=pltpu.CompilerParams(
            dimension_semantics=("parallel", "arbitrary")),
    )(xr, af, w1.astype(cdtype), b1r, w2.astype(cdtype), b2r,
      wr1x, wr1a, br1r, wr2, br2r)

    return out.reshape(B, S, H)
